# Initial kernel scaffold; baseline (speedup 1.0000x reference)
import jax
import jax.numpy as jnp
from jax.experimental import pallas as pl


def kernel(pseudo_boxes, gt_bboxes, merge_boxes):
    raise NotImplementedError("write your pallas kernel here")


# speedup vs baseline: 19.3539x; 19.3539x over previous
import jax
import jax.numpy as jnp
from jax import lax
from jax.experimental import pallas as pl

NUM_GT = 8192
P = 512
R = 256
GRID = NUM_GT // R
NBINS = 50
HPAD = 64


def _iou(ax1, ay1, ax2, ay2, bx1, by1, bx2, by2, eps=1e-6):
    ltx = jnp.maximum(ax1, bx1)
    lty = jnp.maximum(ay1, by1)
    rbx = jnp.minimum(ax2, bx2)
    rby = jnp.minimum(ay2, by2)
    w = jnp.maximum(rbx - ltx, 0.0)
    h = jnp.maximum(rby - lty, 0.0)
    ov = w * h
    a1 = (ax2 - ax1) * (ay2 - ay1)
    a2 = (bx2 - bx1) * (by2 - by1)
    union = jnp.maximum(a1 + a2 - ov, eps)
    return ov / union


def _small_hist(vals, lo, hi):
    b = jnp.floor((vals - lo) * (NBINS / (hi - lo)))
    b = jnp.clip(b, 0.0, NBINS - 1.0).astype(jnp.int32)
    iota = lax.broadcasted_iota(jnp.int32, (1, HPAD), 1)
    eq = jnp.where(b == iota, 1.0, 0.0)
    return jnp.sum(eq, axis=0, keepdims=True)


def _body(x1_ref, y1_ref, x2_ref, y2_ref, gt_ref, mb_ref,
          minus_ref, norm_ref, h1_ref, hmax_ref, hmean_ref, hgt_ref, htop_ref):
    i = pl.program_id(0)

    x1 = x1_ref[...]
    y1 = y1_ref[...]
    x2 = x2_ref[...]
    y2 = y2_ref[...]
    gt = gt_ref[...]
    mb = mb_ref[...]

    g0, g1, g2, g3 = gt[:, 0:1], gt[:, 1:2], gt[:, 2:3], gt[:, 3:4]
    m0, m1, m2, m3 = mb[:, 0:1], mb[:, 1:2], mb[:, 2:3], mb[:, 3:4]

    s = _iou(g0, g1, g2, g3, m0, m1, m2, m3)

    iou1 = _iou(x1, y1, x2, y2, g0, g1, g2, g3)

    minus = iou1 - s
    norm = minus / s
    mask = (iou1 < 0.5) & (s < 0.5)
    minus_ref[...] = jnp.where(mask, 100.0, minus)
    norm_ref[...] = jnp.where(mask, 100.0, norm)

    mean_r = jnp.sum(iou1, axis=1, keepdims=True) * (1.0 / P)

    x = iou1
    acc = jnp.zeros((R, 1), jnp.float32)
    need = jnp.full((R, 1), 10.0, jnp.float32)
    maxv = None
    for k in range(10):
        m = jnp.max(x, axis=1, keepdims=True)
        if k == 0:
            maxv = m
        d = jnp.sum(jnp.where(x == m, 1.0, 0.0), axis=1, keepdims=True)
        take = jnp.minimum(d, need)
        acc += take * m
        need -= take
        if k < 9:
            x = jnp.where(x == m, -1.0, x)
    top10_mean = acc / 10.0

    @pl.when(i == 0)
    def _init():
        zero = jnp.zeros((1, HPAD), jnp.float32)
        h1_ref[...] = zero
        hmax_ref[...] = zero
        hmean_ref[...] = zero
        hgt_ref[...] = zero
        htop_ref[...] = zero

    hmax_ref[...] += _small_hist(maxv, 0.0, 1.0)
    hmean_ref[...] += _small_hist(mean_r, 0.0, 1.0)
    htop_ref[...] += _small_hist(top10_mean, 0.0, 1.0)
    hgt_ref[...] += _small_hist(s, -1.0, 1.0)

    bb = jnp.clip(jnp.floor(iou1 * float(NBINS)), 0.0, NBINS - 1.0).astype(jnp.int32)
    lane_iota = lax.broadcasted_iota(jnp.int32, (1, HPAD), 1)
    acc_h = jnp.zeros((1, HPAD), jnp.float32)
    for k in range(NBINS):
        cnt = jnp.sum(jnp.where(bb == k, 1.0, 0.0))
        acc_h = jnp.where(lane_iota == k, acc_h + cnt, acc_h)
    h1_ref[...] += acc_h


def kernel(pseudo_boxes, gt_bboxes, merge_boxes):
    pb = pseudo_boxes.reshape(NUM_GT, P, 4)
    x1 = pb[:, :, 0]
    y1 = pb[:, :, 1]
    x2 = pb[:, :, 2]
    y2 = pb[:, :, 3]

    f32 = jnp.float32
    outs = pl.pallas_call(
        _body,
        grid=(GRID,),
        in_specs=[
            pl.BlockSpec((R, P), lambda i: (i, 0)),
            pl.BlockSpec((R, P), lambda i: (i, 0)),
            pl.BlockSpec((R, P), lambda i: (i, 0)),
            pl.BlockSpec((R, P), lambda i: (i, 0)),
            pl.BlockSpec((R, 4), lambda i: (i, 0)),
            pl.BlockSpec((R, 4), lambda i: (i, 0)),
        ],
        out_specs=[
            pl.BlockSpec((R, P), lambda i: (i, 0)),
            pl.BlockSpec((R, P), lambda i: (i, 0)),
            pl.BlockSpec((1, HPAD), lambda i: (0, 0)),
            pl.BlockSpec((1, HPAD), lambda i: (0, 0)),
            pl.BlockSpec((1, HPAD), lambda i: (0, 0)),
            pl.BlockSpec((1, HPAD), lambda i: (0, 0)),
            pl.BlockSpec((1, HPAD), lambda i: (0, 0)),
        ],
        out_shape=[
            jax.ShapeDtypeStruct((NUM_GT, P), f32),
            jax.ShapeDtypeStruct((NUM_GT, P), f32),
            jax.ShapeDtypeStruct((1, HPAD), f32),
            jax.ShapeDtypeStruct((1, HPAD), f32),
            jax.ShapeDtypeStruct((1, HPAD), f32),
            jax.ShapeDtypeStruct((1, HPAD), f32),
            jax.ShapeDtypeStruct((1, HPAD), f32),
        ],
    )(x1, y1, x2, y2, gt_bboxes, merge_boxes)

    minus, norm, h1, hmax, hmean, hgt, htop = outs
    return (h1[0, :NBINS], hmax[0, :NBINS], hmean[0, :NBINS],
            hgt[0, :NBINS], htop[0, :NBINS], minus, norm)


# speedup vs baseline: 22.6241x; 1.1690x over previous
import jax
import jax.numpy as jnp
from jax import lax
from jax.experimental import pallas as pl

NUM_GT = 8192
P = 512
R = 256
GRID = NUM_GT // R
NBINS = 50
HPAD = 64


def _iou(ax1, ay1, ax2, ay2, bx1, by1, bx2, by2, eps=1e-6):
    ltx = jnp.maximum(ax1, bx1)
    lty = jnp.maximum(ay1, by1)
    rbx = jnp.minimum(ax2, bx2)
    rby = jnp.minimum(ay2, by2)
    w = jnp.maximum(rbx - ltx, 0.0)
    h = jnp.maximum(rby - lty, 0.0)
    ov = w * h
    a1 = (ax2 - ax1) * (ay2 - ay1)
    a2 = (bx2 - bx1) * (by2 - by1)
    union = jnp.maximum(a1 + a2 - ov, eps)
    return ov / union


def _small_hist(vals, lo, hi):
    b = jnp.floor((vals - lo) * (NBINS / (hi - lo)))
    b = jnp.clip(b, 0.0, NBINS - 1.0).astype(jnp.int32)
    iota = lax.broadcasted_iota(jnp.int32, (1, HPAD), 1)
    eq = jnp.where(b == iota, 1.0, 0.0)
    return jnp.sum(eq, axis=0, keepdims=True)


def _body(x1_ref, y1_ref, x2_ref, y2_ref, gt_ref, mb_ref,
          minus_ref, norm_ref, h1_ref, hmax_ref, hmean_ref, hgt_ref, htop_ref):
    i = pl.program_id(0)

    x1 = x1_ref[...]
    y1 = y1_ref[...]
    x2 = x2_ref[...]
    y2 = y2_ref[...]
    gt = gt_ref[...]
    mb = mb_ref[...]

    g0, g1, g2, g3 = gt[:, 0:1], gt[:, 1:2], gt[:, 2:3], gt[:, 3:4]
    m0, m1, m2, m3 = mb[:, 0:1], mb[:, 1:2], mb[:, 2:3], mb[:, 3:4]

    s = _iou(g0, g1, g2, g3, m0, m1, m2, m3)

    iou1 = _iou(x1, y1, x2, y2, g0, g1, g2, g3)

    minus = iou1 - s
    norm = minus / s
    mask = (iou1 < 0.5) & (s < 0.5)
    minus_ref[...] = jnp.where(mask, 100.0, minus)
    norm_ref[...] = jnp.where(mask, 100.0, norm)

    mean_r = jnp.sum(iou1, axis=1, keepdims=True) * (1.0 / P)

    x = iou1
    acc = jnp.zeros((R, 1), jnp.float32)
    need = jnp.full((R, 1), 10.0, jnp.float32)
    maxv = None
    for k in range(10):
        m = jnp.max(x, axis=1, keepdims=True)
        if k == 0:
            maxv = m
        d = jnp.sum(jnp.where(x == m, 1.0, 0.0), axis=1, keepdims=True)
        take = jnp.minimum(d, need)
        acc += take * m
        need -= take
        if k < 9:
            x = jnp.where(x == m, -1.0, x)
    top10_mean = acc / 10.0

    @pl.when(i == 0)
    def _init():
        zero = jnp.zeros((1, HPAD), jnp.float32)
        h1_ref[...] = zero
        hmax_ref[...] = zero
        hmean_ref[...] = zero
        hgt_ref[...] = zero
        htop_ref[...] = zero

    hmax_ref[...] += _small_hist(maxv, 0.0, 1.0)
    hmean_ref[...] += _small_hist(mean_r, 0.0, 1.0)
    htop_ref[...] += _small_hist(top10_mean, 0.0, 1.0)
    hgt_ref[...] += _small_hist(s, -1.0, 1.0)

    bb = jnp.clip(jnp.floor(iou1 * float(NBINS)), 0.0, NBINS - 1.0).astype(jnp.int32)
    lane_iota = lax.broadcasted_iota(jnp.int32, (1, HPAD), 1)
    acc_h = jnp.zeros((1, HPAD), jnp.float32)
    for k in range(NBINS):
        cnt = jnp.sum(jnp.where(bb == k, 1.0, 0.0))
        acc_h = jnp.where(lane_iota == k, acc_h + cnt, acc_h)
    h1_ref[...] += acc_h


def kernel(pseudo_boxes, gt_bboxes, merge_boxes):
    pb = pseudo_boxes.reshape(NUM_GT, P, 4)
    pbt = jnp.transpose(pb, (2, 0, 1))
    x1, y1, x2, y2 = pbt[0], pbt[1], pbt[2], pbt[3]

    f32 = jnp.float32
    outs = pl.pallas_call(
        _body,
        grid=(GRID,),
        in_specs=[
            pl.BlockSpec((R, P), lambda i: (i, 0)),
            pl.BlockSpec((R, P), lambda i: (i, 0)),
            pl.BlockSpec((R, P), lambda i: (i, 0)),
            pl.BlockSpec((R, P), lambda i: (i, 0)),
            pl.BlockSpec((R, 4), lambda i: (i, 0)),
            pl.BlockSpec((R, 4), lambda i: (i, 0)),
        ],
        out_specs=[
            pl.BlockSpec((R, P), lambda i: (i, 0)),
            pl.BlockSpec((R, P), lambda i: (i, 0)),
            pl.BlockSpec((1, HPAD), lambda i: (0, 0)),
            pl.BlockSpec((1, HPAD), lambda i: (0, 0)),
            pl.BlockSpec((1, HPAD), lambda i: (0, 0)),
            pl.BlockSpec((1, HPAD), lambda i: (0, 0)),
            pl.BlockSpec((1, HPAD), lambda i: (0, 0)),
        ],
        out_shape=[
            jax.ShapeDtypeStruct((NUM_GT, P), f32),
            jax.ShapeDtypeStruct((NUM_GT, P), f32),
            jax.ShapeDtypeStruct((1, HPAD), f32),
            jax.ShapeDtypeStruct((1, HPAD), f32),
            jax.ShapeDtypeStruct((1, HPAD), f32),
            jax.ShapeDtypeStruct((1, HPAD), f32),
            jax.ShapeDtypeStruct((1, HPAD), f32),
        ],
    )(x1, y1, x2, y2, gt_bboxes, merge_boxes)

    minus, norm, h1, hmax, hmean, hgt, htop = outs
    return (h1[0, :NBINS], hmax[0, :NBINS], hmean[0, :NBINS],
            hgt[0, :NBINS], htop[0, :NBINS], minus, norm)
